# Initial kernel scaffold; baseline (speedup 1.0000x reference)
import jax
import jax.numpy as jnp
from jax.experimental import pallas as pl


def kernel(x, edge_index, batch, W1, b1, W2, b2, Wl, bl):
    raise NotImplementedError("write your pallas kernel here")


# speedup vs baseline: 46.8072x; 46.8072x over previous
import functools

import jax
import jax.numpy as jnp
from jax import lax
from jax.experimental import pallas as pl
from jax.experimental.pallas import tpu as pltpu
from jax.experimental.pallas import tpu_sc as plsc

N = 10000
E = 320000
G = 128
DIN = 136
DP = 64
NC = 2
NS = 16
NW = NC * NS
EPW = E // NW
CHUNK = 128
NCHUNK = EPW // CHUNK
TAIL = EPW - NCHUNK * CHUNK
STG = 624
LAST = N - STG * (NS - 1)
RB = 2000
GRID = N // RB

_MESH = plsc.VectorSubcoreMesh(core_axis_name="c", subcore_axis_name="s")


def _split_rows(body):
    sid = lax.axis_index("s")
    r0 = pl.multiple_of(sid * STG, 8)

    @pl.when(sid < NS - 1)
    def _():
        body(r0, STG)

    @pl.when(sid == NS - 1)
    def _():
        body(N - LAST, LAST)


DEG_GRP = 6
NGRP_D = NCHUNK // DEG_GRP
NPAIR_D = (NGRP_D - 1) // 2


@functools.partial(
    pl.kernel,
    out_type=jax.ShapeDtypeStruct((NC, N, 8), jnp.float32),
    mesh=_MESH,
    scratch_types=[
        [pltpu.VMEM((CHUNK,), jnp.int32) for _ in range(2 * DEG_GRP)],
        pltpu.VMEM((TAIL,), jnp.int32),
        pltpu.VMEM((CHUNK, 8), jnp.float32),
        pltpu.VMEM_SHARED((N, 8), jnp.float32),
        pltpu.SemaphoreType.DMA,
        [pltpu.SemaphoreType.DMA for _ in range(2 * DEG_GRP)],
    ],
    compiler_params=pltpu.CompilerParams(use_tc_tiling_on_sc=False),
)
def _deg_sc(ei_hbm, ones_hbm, zeros_hbm, out_hbm, didx, didx_t, ones_v,
            deg_sh, sem, isems):
    cid = lax.axis_index("c")
    sid = lax.axis_index("s")
    wid = cid * NS + sid
    ebase = wid * EPW
    _split_rows(lambda r0, n: pltpu.sync_copy(
        zeros_hbm.at[pl.ds(0, n), :], deg_sh.at[pl.ds(r0, n), :]))
    pltpu.sync_copy(ones_hbm, ones_v)
    plsc.subcore_barrier()

    def fire_idx(grp, s):
        for k in range(DEG_GRP):
            b = s * DEG_GRP + k
            pltpu.async_copy(
                ei_hbm.at[1, pl.ds(ebase + (grp * DEG_GRP + k) * CHUNK,
                                   CHUNK)], didx[b], isems[b])

    def scatter_idx_group(grp, s):
        descs = []
        for k in range(DEG_GRP):
            b = s * DEG_GRP + k
            pltpu.make_async_copy(
                ei_hbm.at[1, pl.ds(ebase + (grp * DEG_GRP + k) * CHUNK,
                                   CHUNK)], didx[b], isems[b]).wait()
            descs.append(pltpu.async_copy(ones_v, deg_sh.at[didx[b]],
                                          sem, add=True))
        return descs

    fire_idx(0, 0)

    def step(j, carry):
        ga = 2 * j
        fire_idx(ga + 1, 1)
        for d in scatter_idx_group(ga, 0):
            d.wait()
        fire_idx(ga + 2, 0)
        for d in scatter_idx_group(ga + 1, 1):
            d.wait()
        return carry

    lax.fori_loop(0, NPAIR_D, step, 0)
    for d in scatter_idx_group(NGRP_D - 1, 0):
        d.wait()
    pltpu.sync_copy(ei_hbm.at[1, pl.ds(ebase + NCHUNK * CHUNK, TAIL)],
                    didx_t)
    pltpu.async_copy(ones_v.at[pl.ds(0, TAIL), :], deg_sh.at[didx_t],
                     sem, add=True).wait()
    plsc.subcore_barrier()
    _split_rows(lambda r0, n: pltpu.sync_copy(
        deg_sh.at[pl.ds(r0, n), :], out_hbm.at[cid, pl.ds(r0, n), :]))


NBUF = 3
NGRP = NCHUNK // NBUF
NPAIR = (NGRP - 2) // 2


@functools.partial(
    pl.kernel,
    out_type=jax.ShapeDtypeStruct((NC, N, DP), jnp.float32),
    mesh=_MESH,
    scratch_types=[
        pltpu.VMEM((EPW,), jnp.int32),
        [pltpu.VMEM((CHUNK,), jnp.int32) for _ in range(2 * NBUF)],
        pltpu.VMEM((TAIL,), jnp.int32),
        [pltpu.VMEM((CHUNK, DP), jnp.float32) for _ in range(2 * NBUF)],
        pltpu.VMEM((TAIL, DP), jnp.float32),
        pltpu.VMEM_SHARED((N, DP), jnp.float32),
        [pltpu.SemaphoreType.DMA for _ in range(2 * NBUF)],
        [pltpu.SemaphoreType.DMA for _ in range(2 * NBUF)],
        [pltpu.SemaphoreType.DMA for _ in range(2 * NBUF)],
    ],
    compiler_params=pltpu.CompilerParams(use_tc_tiling_on_sc=False),
)
def _agg_sc(hp_hbm, ei_hbm, zeros_hbm, out_hbm,
            sidx_all, didx, didx_t, rows, rows_t, acc_sh,
            gsems, ssems, isems):
    cid = lax.axis_index("c")
    sid = lax.axis_index("s")
    wid = cid * NS + sid
    ebase = wid * EPW

    pltpu.sync_copy(ei_hbm.at[0, pl.ds(ebase, EPW)], sidx_all)

    @pl.when(cid == 0)
    def _():
        _split_rows(lambda r0, n: pltpu.sync_copy(
            hp_hbm.at[pl.ds(r0, n), :], acc_sh.at[pl.ds(r0, n), :]))

    @pl.when(cid != 0)
    def _():
        _split_rows(lambda r0, n: pltpu.sync_copy(
            zeros_hbm.at[pl.ds(0, n), :], acc_sh.at[pl.ds(r0, n), :]))

    plsc.subcore_barrier()

    def fire_gathers(grp, s):
        for b in range(NBUF):
            i = s * NBUF + b
            off = (grp * NBUF + b) * CHUNK
            pltpu.async_copy(ei_hbm.at[1, pl.ds(ebase + off, CHUNK)],
                             didx[i], isems[i])
            pltpu.async_copy(hp_hbm.at[sidx_all.at[pl.ds(off, CHUNK)]],
                             rows[i], gsems[i])

    def scatter_group(grp, s):
        sds = []
        for b in range(NBUF):
            i = s * NBUF + b
            off = (grp * NBUF + b) * CHUNK
            pltpu.make_async_copy(ei_hbm.at[1, pl.ds(ebase + off, CHUNK)],
                                  didx[i], isems[i]).wait()
            pltpu.make_async_copy(hp_hbm.at[sidx_all.at[pl.ds(off, CHUNK)]],
                                  rows[i], gsems[i]).wait()
            sds.append(pltpu.async_copy(rows[i], acc_sh.at[didx[i]],
                                        ssems[i], add=True))
        return sds

    fire_gathers(0, 0)

    def step(k, carry):
        ga = 2 * k
        fire_gathers(ga + 1, 1)
        sds_a = scatter_group(ga, 0)
        for d in sds_a:
            d.wait()
        fire_gathers(ga + 2, 0)
        sds_b = scatter_group(ga + 1, 1)
        for d in sds_b:
            d.wait()
        return carry

    lax.fori_loop(0, NPAIR, step, 0)
    fire_gathers(NGRP - 1, 1)
    for d in scatter_group(NGRP - 2, 0):
        d.wait()
    for d in scatter_group(NGRP - 1, 1):
        d.wait()
    toff = NCHUNK * CHUNK
    pltpu.sync_copy(ei_hbm.at[1, pl.ds(ebase + toff, TAIL)], didx_t)
    pltpu.async_copy(hp_hbm.at[sidx_all.at[pl.ds(toff, TAIL)]],
                     rows_t, gsems[0]).wait()
    pltpu.async_copy(rows_t, acc_sh.at[didx_t], ssems[0], add=True).wait()

    plsc.subcore_barrier()
    _split_rows(lambda r0, n: pltpu.sync_copy(
        acc_sh.at[pl.ds(r0, n), :], out_hbm.at[cid, pl.ds(r0, n), :]))


def _dinv_of(deg_ref):
    cnt = deg_ref[0, :, 0] + deg_ref[1, :, 0] + 1.0
    return lax.rsqrt(cnt)[:, None]


def _mm1_body(x_ref, w_ref, o_ref):
    o_ref[...] = jnp.dot(x_ref[...], w_ref[...],
                         preferred_element_type=jnp.float32)


def _scale_body(h_ref, deg_ref, o_ref):
    o_ref[...] = _dinv_of(deg_ref) * h_ref[...]


def _mid_body(acc_ref, deg_ref, b_ref, w_ref, o_ref):
    dinv = _dinv_of(deg_ref)
    out1 = jnp.maximum(dinv * (acc_ref[0] + acc_ref[1]) + b_ref[...], 0.0)
    o_ref[...] = dinv * jnp.dot(out1, w_ref[...],
                                preferred_element_type=jnp.float32)


def _fin_body(acc_ref, deg_ref, b_ref, batch_ref, wl_ref, bl_ref,
              sums_ref, out_ref):
    i = pl.program_id(0)
    dinv = _dinv_of(deg_ref)
    h2 = jnp.maximum(dinv * (acc_ref[0] + acc_ref[1]) + b_ref[...], 0.0)
    ones_col = (lax.broadcasted_iota(jnp.int32, (RB, DP), 1) == DP - 1)
    h2 = h2 + ones_col.astype(jnp.float32)
    seg = batch_ref[0]
    gids = lax.broadcasted_iota(jnp.int32, (G, RB), 0)
    onehot = (gids == jnp.broadcast_to(seg, (G, RB))).astype(jnp.float32)
    psum = jnp.dot(onehot, h2, preferred_element_type=jnp.float32)

    @pl.when(i == 0)
    def _():
        sums_ref[...] = jnp.zeros_like(sums_ref)

    sums_ref[...] += psum

    @pl.when(i == pl.num_programs(0) - 1)
    def _():
        cnt = jnp.maximum(sums_ref[:, DP - 1:DP], 1.0)
        pooled = sums_ref[...] / cnt
        z = jnp.dot(pooled, wl_ref[...],
                    preferred_element_type=jnp.float32) + bl_ref[0, 0]
        out_ref[...] = 1.0 / (1.0 + jnp.exp(-z))


def kernel(x, edge_index, batch, W1, b1, W2, b2, Wl, bl):
    f32 = jnp.float32
    W1p = jnp.pad(W1, ((0, 0), (0, DP - W1.shape[1])))
    W2p = jnp.pad(W2, ((0, DP - W2.shape[0]), (0, DP - W2.shape[1])))
    b1p = jnp.pad(b1, (0, DP - b1.shape[0]))[None, :]
    b2p = jnp.pad(b2, (0, DP - b2.shape[0]))[None, :]
    Wlp = jnp.pad(Wl, ((0, DP - Wl.shape[0]), (0, 128 - Wl.shape[1])))
    blv = bl.reshape(1, 1)
    ones8 = jnp.ones((CHUNK, 8), f32)
    zeros8 = jnp.zeros((LAST, 8), f32)
    zeros64 = jnp.zeros((LAST, DP), f32)
    batch3 = batch.reshape(GRID, 1, RB)

    h1raw = pl.pallas_call(
        _mm1_body,
        grid=(GRID,),
        in_specs=[
            pl.BlockSpec((RB, DIN), lambda i: (i, 0)),
            pl.BlockSpec((DIN, DP), lambda i: (0, 0)),
        ],
        out_specs=pl.BlockSpec((RB, DP), lambda i: (i, 0)),
        out_shape=jax.ShapeDtypeStruct((N, DP), f32),
    )(x, W1p)

    deg2 = _deg_sc(edge_index, ones8, zeros8)

    h1p = pl.pallas_call(
        _scale_body,
        grid=(GRID,),
        in_specs=[
            pl.BlockSpec((RB, DP), lambda i: (i, 0)),
            pl.BlockSpec((NC, RB, 8), lambda i: (0, i, 0)),
        ],
        out_specs=pl.BlockSpec((RB, DP), lambda i: (i, 0)),
        out_shape=jax.ShapeDtypeStruct((N, DP), f32),
    )(h1raw, deg2)

    a1 = _agg_sc(h1p, edge_index, zeros64)

    h2p = pl.pallas_call(
        _mid_body,
        grid=(GRID,),
        in_specs=[
            pl.BlockSpec((NC, RB, DP), lambda i: (0, i, 0)),
            pl.BlockSpec((NC, RB, 8), lambda i: (0, i, 0)),
            pl.BlockSpec((1, DP), lambda i: (0, 0)),
            pl.BlockSpec((DP, DP), lambda i: (0, 0)),
        ],
        out_specs=pl.BlockSpec((RB, DP), lambda i: (i, 0)),
        out_shape=jax.ShapeDtypeStruct((N, DP), f32),
    )(a1, deg2, b1p, W2p)

    a2 = _agg_sc(h2p, edge_index, zeros64)

    _, out = pl.pallas_call(
        _fin_body,
        grid=(GRID,),
        in_specs=[
            pl.BlockSpec((NC, RB, DP), lambda i: (0, i, 0)),
            pl.BlockSpec((NC, RB, 8), lambda i: (0, i, 0)),
            pl.BlockSpec((1, DP), lambda i: (0, 0)),
            pl.BlockSpec((1, 1, RB), lambda i: (i, 0, 0)),
            pl.BlockSpec((DP, 128), lambda i: (0, 0)),
            pl.BlockSpec((1, 1), lambda i: (0, 0)),
        ],
        out_specs=[
            pl.BlockSpec((G, DP), lambda i: (0, 0)),
            pl.BlockSpec((G, 128), lambda i: (0, 0)),
        ],
        out_shape=[
            jax.ShapeDtypeStruct((G, DP), f32),
            jax.ShapeDtypeStruct((G, 128), f32),
        ],
    )(a2, deg2, b2p, batch3, Wlp, blv)

    return out[:, 0].reshape(-1)
